# baseline (device time: 170360 ns/iter reference)
import jax
import jax.numpy as jnp
from jax import lax
from jax.experimental import pallas as pl
from jax.experimental.pallas import tpu as pltpu

N_DEV = 4
CH = 1024
HF = CH // 2
SUB = 2
SS = HF // SUB


def kernel(x, W1, W2):
    m, k = x.shape
    d = W1.shape[1]
    n = W2.shape[1]

    def body(x_ref, w1_ref, w2_ref, out_ref, cw_ref, ccw_ref,
             tA_ref, tB_ref,
             cw_send_sems, cw_recv_sems, ccw_send_sems, ccw_recv_sems,
             agcw_send_sems, agcw_recv_sems, agccw_send_sems,
             agccw_recv_sems):
        my = lax.axis_index("i")
        left = (my - 1) % N_DEV
        right = (my + 1) % N_DEV

        barrier_sem = pltpu.get_barrier_semaphore()
        for nbr in [left, right]:
            pl.semaphore_signal(
                barrier_sem, inc=1,
                device_id=(nbr,), device_id_type=pl.DeviceIdType.MESH,
            )
        pl.semaphore_wait(barrier_sem, 2)

        def gemm1_top(c, u):
            return jnp.dot(
                x_ref[pl.ds(c * CH + u * SS, SS), :], w1_ref[:, :],
                preferred_element_type=jnp.float32,
            )

        def gemm1_bot(c, u):
            return jnp.dot(
                x_ref[pl.ds(c * CH + HF + u * SS, SS), :], w1_ref[:, :],
                preferred_element_type=jnp.float32,
            )

        def gemm2_top(c, u, buf):
            out_ref[pl.ds(c * CH + u * SS, SS), :] = jnp.dot(
                buf, w2_ref[:, :], preferred_element_type=jnp.float32,
            )

        def gemm2_bot(c, u, buf):
            out_ref[pl.ds(c * CH + HF + u * SS, SS), :] = jnp.dot(
                buf, w2_ref[:, :], preferred_element_type=jnp.float32,
            )

        def rdma(src, dst, ssem, rsem, target):
            return pltpu.make_async_remote_copy(
                src_ref=src, dst_ref=dst, send_sem=ssem, recv_sem=rsem,
                device_id=(target,), device_id_type=pl.DeviceIdType.MESH,
            )

        def slot(s, u):
            return s * SUB + u

        rs_cw = {}
        rs_ccw = {}

        def make_rs(s, u):
            rs_cw[(s, u)] = rdma(
                tA_ref.at[u] if s == 0 else cw_ref.at[slot(s - 1, u)],
                cw_ref.at[slot(s, u)],
                cw_send_sems.at[slot(s, u)], cw_recv_sems.at[slot(s, u)],
                right)
            rs_ccw[(s, u)] = rdma(
                tB_ref.at[u] if s == 0 else ccw_ref.at[slot(s - 1, u)],
                ccw_ref.at[slot(s, u)],
                ccw_send_sems.at[slot(s, u)], ccw_recv_sems.at[slot(s, u)],
                left)

        for u in range(SUB):
            tA_ref[u, :, :] = gemm1_top(my, u)
            tB_ref[u, :, :] = gemm1_bot(my, u)
            make_rs(0, u)
            rs_cw[(0, u)].start()
            rs_ccw[(0, u)].start()

        ag_base = [3, 0, 1]
        ag_cw = {}
        ag_ccw = {}

        def make_ag(s, u):
            ag_cw[(s, u)] = rdma(
                cw_ref.at[slot(2, u)] if s == 0
                else cw_ref.at[slot(ag_base[s - 1], u)],
                cw_ref.at[slot(ag_base[s], u)],
                agcw_send_sems.at[slot(s, u)], agcw_recv_sems.at[slot(s, u)],
                right)
            ag_ccw[(s, u)] = rdma(
                ccw_ref.at[slot(2, u)] if s == 0
                else ccw_ref.at[slot(ag_base[s - 1], u)],
                ccw_ref.at[slot(ag_base[s], u)],
                agccw_send_sems.at[slot(s, u)],
                agccw_recv_sems.at[slot(s, u)],
                left)

        for s in range(3):
            for u in range(SUB):
                rs_cw[(s, u)].wait()
                cw_ref[slot(s, u), :, :] = (
                    cw_ref[slot(s, u), :, :] + gemm1_top((my - s - 1) % N_DEV, u)
                )
                if s < 2:
                    make_rs(s + 1, u)
                    rs_cw[(s + 1, u)].start()
                else:
                    make_ag(0, u)
                    ag_cw[(0, u)].start()
                rs_ccw[(s, u)].wait()
                ccw_ref[slot(s, u), :, :] = (
                    ccw_ref[slot(s, u), :, :] + gemm1_bot((my + s + 1) % N_DEV, u)
                )
                if s < 2:
                    rs_ccw[(s + 1, u)].start()
                else:
                    ag_ccw[(0, u)].start()
                    gemm2_top((my + 1) % N_DEV, u, cw_ref[slot(2, u), :, :])
                    gemm2_bot((my - 1) % N_DEV, u, ccw_ref[slot(2, u), :, :])

        for s in range(3):
            for u in range(SUB):
                ag_cw[(s, u)].wait()
                if s < 2:
                    make_ag(s + 1, u)
                    ag_cw[(s + 1, u)].start()
                gemm2_top((my - s) % N_DEV, u,
                          cw_ref[slot(ag_base[s], u), :, :])
                ag_ccw[(s, u)].wait()
                if s < 2:
                    ag_ccw[(s + 1, u)].start()
                gemm2_bot((my + s) % N_DEV, u,
                          ccw_ref[slot(ag_base[s], u), :, :])

    return pl.pallas_call(
        body,
        out_shape=jax.ShapeDtypeStruct((m, n), jnp.float32),
        in_specs=[
            pl.BlockSpec(memory_space=pltpu.VMEM),
            pl.BlockSpec(memory_space=pltpu.VMEM),
            pl.BlockSpec(memory_space=pltpu.VMEM),
        ],
        out_specs=pl.BlockSpec(memory_space=pltpu.VMEM),
        scratch_shapes=[
            pltpu.VMEM((4 * SUB, SS, d), jnp.float32),
            pltpu.VMEM((4 * SUB, SS, d), jnp.float32),
            pltpu.VMEM((SUB, SS, d), jnp.float32),
            pltpu.VMEM((SUB, SS, d), jnp.float32),
            pltpu.SemaphoreType.DMA((3 * SUB,)),
            pltpu.SemaphoreType.DMA((3 * SUB,)),
            pltpu.SemaphoreType.DMA((3 * SUB,)),
            pltpu.SemaphoreType.DMA((3 * SUB,)),
            pltpu.SemaphoreType.DMA((3 * SUB,)),
            pltpu.SemaphoreType.DMA((3 * SUB,)),
            pltpu.SemaphoreType.DMA((3 * SUB,)),
            pltpu.SemaphoreType.DMA((3 * SUB,)),
        ],
        compiler_params=pltpu.CompilerParams(
            collective_id=0,
            vmem_limit_bytes=64 * 1024 * 1024,
        ),
    )(x, W1, W2)
